# baseline (speedup 1.0000x reference)
import jax
import jax.numpy as jnp
from jax.experimental import pallas as pl
from jax.experimental.pallas import tpu as pltpu


def _gcn_kernel(x_ref, a_ref, w1_ref, b1_ref, w2_ref, b2_ref, out_ref,
                y_s, g_s):
    p = pl.program_id(0)
    i = pl.program_id(1)
    bm = x_ref.shape[0]

    @pl.when(p == 0)
    def _():
        y_s[pl.ds(i * bm, bm), :] = jnp.dot(
            x_ref[...], w1_ref[...], preferred_element_type=jnp.float32)

    @pl.when(p == 1)
    def _():
        h = jnp.dot(a_ref[...], y_s[...], preferred_element_type=jnp.float32)
        h = jnp.maximum(h + b1_ref[...], 0.0)
        g_s[pl.ds(i * bm, bm), :] = jnp.dot(
            h, w2_ref[...], preferred_element_type=jnp.float32)

    @pl.when(p == 2)
    def _():
        out_ref[...] = jnp.dot(
            a_ref[...], g_s[...], preferred_element_type=jnp.float32
        ) + b2_ref[...]


def kernel(X, A, W1, b1, W2, b2):
    V, cin = X.shape
    nh = W1.shape[1]
    cout = W2.shape[1]
    bm = 400
    nb = V // bm

    return pl.pallas_call(
        _gcn_kernel,
        grid=(3, nb),
        in_specs=[
            pl.BlockSpec((bm, cin), lambda p, i: (jnp.where(p == 0, i, 0), 0)),
            pl.BlockSpec((bm, V), lambda p, i: (jnp.where(p == 0, 0, i), 0)),
            pl.BlockSpec((cin, nh), lambda p, i: (0, 0)),
            pl.BlockSpec((1, nh), lambda p, i: (0, 0)),
            pl.BlockSpec((nh, cout), lambda p, i: (0, 0)),
            pl.BlockSpec((1, cout), lambda p, i: (0, 0)),
        ],
        out_specs=pl.BlockSpec((bm, cout), lambda p, i: (jnp.where(p == 2, i, 0), 0)),
        out_shape=jax.ShapeDtypeStruct((V, cout), jnp.float32),
        scratch_shapes=[
            pltpu.VMEM((V, nh), jnp.float32),
            pltpu.VMEM((V, cout), jnp.float32),
        ],
    )(X, A, W1, b1.reshape(1, -1), W2, b2.reshape(1, -1))
